# baseline (speedup 1.0000x reference)
import functools
import math

import jax
import jax.numpy as jnp
from jax import lax
from jax.experimental import pallas as pl
from jax.experimental.pallas import tpu as pltpu
from jax.experimental.pallas import tpu_sc as plsc


def _make_sc_kernel(N, D, scale):
    info = plsc.get_sparse_core_info()
    NC, NS, L = info.num_cores, info.num_subcores, info.num_lanes
    NW = NC * NS
    per_w = N // NW
    C = 64
    n_chunks = per_w // C
    mesh = plsc.VectorSubcoreMesh(core_axis_name="c", subcore_axis_name="s")

    @functools.partial(
        pl.kernel,
        mesh=mesh,
        out_type=jax.ShapeDtypeStruct((N, D), jnp.float32),
        scratch_types=[
            pltpu.VMEM((C,), jnp.int32),
            pltpu.VMEM((C, D), jnp.float32),
            pltpu.SemaphoreType.DMA,
        ],
    )
    def k(x_hbm, lut_hbm, out_hbm, idx_v, rows_v, sem):
        wid = lax.axis_index("s") * NC + lax.axis_index("c")
        base = wid * per_w

        def chunk_body(c, carry):
            row0 = base + c * C
            pltpu.sync_copy(x_hbm.at[pl.ds(row0, C)], idx_v)
            pltpu.async_copy(lut_hbm.at[idx_v], rows_v, sem).wait()

            def row_body(r, carry2):
                def vec_body(j, carry3):
                    sl = pl.ds(j * L, L)
                    rows_v[r, sl] = rows_v[r, sl] * scale
                    return carry3
                return lax.fori_loop(0, D // L, vec_body, carry2)

            lax.fori_loop(0, C, row_body, 0)
            pltpu.sync_copy(rows_v, out_hbm.at[pl.ds(row0, C), :])
            return carry

        lax.fori_loop(0, n_chunks, chunk_body, 0)

    return k


def kernel(x, lut):
    B, S = x.shape
    _, D = lut.shape
    N = B * S
    scale = float(math.sqrt(D))
    xf = x.reshape(N).astype(jnp.int32)
    out = _make_sc_kernel(N, D, scale)(xf, lut)
    return out.reshape(B, S, D)


# speedup vs baseline: 2.8522x; 2.8522x over previous
import functools
import math

import jax
import jax.numpy as jnp
from jax import lax
from jax.experimental import pallas as pl
from jax.experimental.pallas import tpu as pltpu
from jax.experimental.pallas import tpu_sc as plsc


def _make_sc_kernel(N, D, scale):
    info = plsc.get_sparse_core_info()
    NC, NS, L = info.num_cores, info.num_subcores, info.num_lanes
    NW = NC * NS
    per_w = N // NW
    C = 32
    n_chunks = per_w // C
    mesh = plsc.VectorSubcoreMesh(core_axis_name="c", subcore_axis_name="s")

    @functools.partial(
        pl.kernel,
        mesh=mesh,
        out_type=jax.ShapeDtypeStruct((N, D), jnp.float32),
        scratch_types=[
            pltpu.VMEM((n_chunks, C), jnp.int32),
            pltpu.VMEM((C, D), jnp.float32),
            pltpu.VMEM((C, D), jnp.float32),
            pltpu.SemaphoreType.DMA,
            pltpu.SemaphoreType.DMA,
            pltpu.SemaphoreType.DMA,
            pltpu.SemaphoreType.DMA,
        ],
    )
    def k(x_hbm, lut_hbm, out_hbm, idx_all, rows0, rows1, g0, g1, s0, s1):
        wid = lax.axis_index("s") * NC + lax.axis_index("c")
        base = wid * per_w
        rows = (rows0, rows1)
        gsem = (g0, g1)
        ssem = (s0, s1)

        pltpu.sync_copy(x_hbm.at[wid], idx_all)

        def gather(c):
            b = c % 2
            return pltpu.async_copy(lut_hbm.at[idx_all.at[c]], rows[b], gsem[b])

        def scatter(c):
            b = c % 2
            return pltpu.async_copy(
                rows[b], out_hbm.at[pl.ds(base + c * C, C), :], ssem[b])

        def scale_buf(rv):
            def row_body(r, carry):
                for j in range(D // L):
                    sl = pl.ds(j * L, L)
                    rv[r, sl] = rv[r, sl] * scale
                return carry
            lax.fori_loop(0, C, row_body, 0)

        h_g = [None] * n_chunks
        h_s = [None] * n_chunks
        h_g[0] = gather(0)
        for c in range(n_chunks):
            if c + 1 < n_chunks:
                if c - 1 >= 0:
                    h_s[c - 1].wait()
                h_g[c + 1] = gather(c + 1)
            h_g[c].wait()
            scale_buf(rows[c % 2])
            h_s[c] = scatter(c)
        h_s[n_chunks - 2].wait()
        h_s[n_chunks - 1].wait()

    return k


def kernel(x, lut):
    B, S = x.shape
    _, D = lut.shape
    N = B * S
    info = plsc.get_sparse_core_info()
    NW = info.num_cores * info.num_subcores
    per_w = N // NW
    C = 32
    scale = float(math.sqrt(D))
    xf = x.reshape(NW, per_w // C, C).astype(jnp.int32)
    out = _make_sc_kernel(N, D, scale)(xf, lut)
    return out.reshape(B, S, D)


# speedup vs baseline: 2.8647x; 1.0044x over previous
import functools
import math

import jax
import jax.numpy as jnp
from jax import lax
from jax.experimental import pallas as pl
from jax.experimental.pallas import tpu as pltpu
from jax.experimental.pallas import tpu_sc as plsc


def _make_sc_kernel(N, D, scale):
    info = plsc.get_sparse_core_info()
    NC, NS, L = info.num_cores, info.num_subcores, info.num_lanes
    NW = NC * NS
    per_w = N // NW
    C = 32
    n_chunks = per_w // C
    mesh = plsc.VectorSubcoreMesh(core_axis_name="c", subcore_axis_name="s")

    @functools.partial(
        pl.kernel,
        mesh=mesh,
        out_type=jax.ShapeDtypeStruct((N, D), jnp.float32),
        scratch_types=[
            pltpu.VMEM((n_chunks, C), jnp.int32),
            pltpu.VMEM((C, D), jnp.float32),
            pltpu.VMEM((C, D), jnp.float32),
            pltpu.VMEM((C, D), jnp.float32),
            pltpu.SemaphoreType.DMA,
            pltpu.SemaphoreType.DMA,
            pltpu.SemaphoreType.DMA,
            pltpu.SemaphoreType.DMA,
            pltpu.SemaphoreType.DMA,
            pltpu.SemaphoreType.DMA,
        ],
    )
    def k(x_hbm, lut_hbm, out_hbm, idx_all,
          rows0, rows1, rows2, g0, g1, g2, s0, s1, s2):
        wid = lax.axis_index("s") * NC + lax.axis_index("c")
        base = wid * per_w
        rows = (rows0, rows1, rows2)
        gsem = (g0, g1, g2)
        ssem = (s0, s1, s2)
        NB = 3

        pltpu.sync_copy(x_hbm.at[wid], idx_all)

        def gather(c):
            b = c % NB
            return pltpu.async_copy(lut_hbm.at[idx_all.at[c]], rows[b], gsem[b])

        def scatter(c):
            b = c % NB
            return pltpu.async_copy(
                rows[b], out_hbm.at[pl.ds(base + c * C, C), :], ssem[b])

        def scale_buf(rv):
            def row_body(r, carry):
                for j in range(D // L):
                    sl = pl.ds(j * L, L)
                    rv[r, sl] = rv[r, sl] * scale
                return carry
            lax.fori_loop(0, C, row_body, 0)

        h_g = [None] * n_chunks
        h_s = [None] * n_chunks
        h_g[0] = gather(0)
        h_g[1] = gather(1)
        for c in range(n_chunks):
            if c + 2 < n_chunks:
                if c - 1 >= 0:
                    h_s[c - 1].wait()
                h_g[c + 2] = gather(c + 2)
            h_g[c].wait()
            scale_buf(rows[c % NB])
            h_s[c] = scatter(c)
        h_s[n_chunks - 3].wait()
        h_s[n_chunks - 2].wait()
        h_s[n_chunks - 1].wait()

    return k


def kernel(x, lut):
    B, S = x.shape
    _, D = lut.shape
    N = B * S
    info = plsc.get_sparse_core_info()
    NW = info.num_cores * info.num_subcores
    per_w = N // NW
    C = 32
    scale = float(math.sqrt(D))
    xf = x.reshape(NW, per_w // C, C).astype(jnp.int32)
    out = _make_sc_kernel(N, D, scale)(xf, lut)
    return out.reshape(B, S, D)


# speedup vs baseline: 2.9239x; 1.0207x over previous
import functools
import math

import jax
import jax.numpy as jnp
from jax import lax
from jax.experimental import pallas as pl
from jax.experimental.pallas import tpu as pltpu
from jax.experimental.pallas import tpu_sc as plsc


def _make_sc_kernel(N, D, scale):
    info = plsc.get_sparse_core_info()
    NC, NS, L = info.num_cores, info.num_subcores, info.num_lanes
    NW = NC * NS
    per_w = N // NW
    C = 32
    n_chunks = per_w // C
    mesh = plsc.VectorSubcoreMesh(core_axis_name="c", subcore_axis_name="s")

    @functools.partial(
        pl.kernel,
        mesh=mesh,
        out_type=jax.ShapeDtypeStruct((N, D), jnp.float32),
        scratch_types=[
            pltpu.VMEM((n_chunks, C), jnp.int32),
            pltpu.VMEM((C, D), jnp.float32),
            pltpu.VMEM((C, D), jnp.float32),
            pltpu.VMEM((C, D), jnp.float32),
            pltpu.SemaphoreType.DMA,
            pltpu.SemaphoreType.DMA,
            pltpu.SemaphoreType.DMA,
            pltpu.SemaphoreType.DMA,
            pltpu.SemaphoreType.DMA,
            pltpu.SemaphoreType.DMA,
        ],
    )
    def k(x_hbm, lut_hbm, out_hbm, idx_all,
          rows0, rows1, rows2, g0, g1, g2, s0, s1, s2):
        wid = lax.axis_index("s") * NC + lax.axis_index("c")
        base = wid * per_w
        rows = (rows0, rows1, rows2)
        gsem = (g0, g1, g2)
        ssem = (s0, s1, s2)
        NB = 3

        pltpu.sync_copy(x_hbm.at[wid], idx_all)

        def gather(c):
            b = c % NB
            return pltpu.async_copy(lut_hbm.at[idx_all.at[c]], rows[b], gsem[b])

        GR = 8
        n_groups = C // GR

        def scatter_group(c, g):
            b = c % NB
            return pltpu.async_copy(
                rows[b].at[pl.ds(g * GR, GR), :],
                out_hbm.at[pl.ds(base + c * C + g * GR, GR), :],
                ssem[b])

        def scale_group(rv, g):
            def row_body(r, carry):
                for j in range(D // L):
                    sl = pl.ds(j * L, L)
                    rv[r, sl] = rv[r, sl] * scale
                return carry
            lax.fori_loop(g * GR, (g + 1) * GR, row_body, 0)

        h_g = [None] * n_chunks
        h_s = [[None] * n_groups for _ in range(n_chunks)]
        h_g[0] = gather(0)
        h_g[1] = gather(1)
        for c in range(n_chunks):
            if c + 2 < n_chunks:
                if c - 1 >= 0:
                    for hh in h_s[c - 1]:
                        hh.wait()
                h_g[c + 2] = gather(c + 2)
            h_g[c].wait()
            for g in range(n_groups):
                scale_group(rows[c % NB], g)
                h_s[c][g] = scatter_group(c, g)
        for c in (n_chunks - 3, n_chunks - 2, n_chunks - 1):
            for hh in h_s[c]:
                hh.wait()

    return k


def kernel(x, lut):
    B, S = x.shape
    _, D = lut.shape
    N = B * S
    info = plsc.get_sparse_core_info()
    NW = info.num_cores * info.num_subcores
    per_w = N // NW
    C = 32
    scale = float(math.sqrt(D))
    xf = x.reshape(NW, per_w // C, C).astype(jnp.int32)
    out = _make_sc_kernel(N, D, scale)(xf, lut)
    return out.reshape(B, S, D)
